# baseline (speedup 1.0000x reference)
import functools

import jax
import jax.numpy as jnp
from jax import lax
from jax.experimental import pallas as pl
from jax.experimental.pallas import tpu as pltpu
from jax.experimental.pallas import tpu_sc as plsc

_LANES = 16


@functools.lru_cache(maxsize=None)
def _build(B, T, D, NC, NS):
    NW = NC * NS
    BW = B // NW
    RPS = 128
    NCHK = BW // RPS
    OT = T + 1
    NSTEP = OT * NCHK
    KD = D // _LANES
    mesh = plsc.VectorSubcoreMesh(core_axis_name="c", subcore_axis_name="s")

    @functools.partial(
        pl.kernel,
        out_type=jax.ShapeDtypeStruct((OT * B, D), jnp.float32),
        mesh=mesh,
        scratch_types=[
            pltpu.VMEM((NCHK, RPS), jnp.int32),
            pltpu.VMEM((T * NCHK, RPS), jnp.int32),
            pltpu.VMEM((8, D), jnp.float32),
            pltpu.VMEM((8, D), jnp.float32),
            pltpu.VMEM((T, D), jnp.float32),
            pltpu.VMEM((5, RPS, D), jnp.float32),
            pltpu.SemaphoreType.DMA,
            pltpu.SemaphoreType.DMA,
            pltpu.SemaphoreType.DMA,
            pltpu.SemaphoreType.DMA,
            pltpu.SemaphoreType.DMA,
            pltpu.SemaphoreType.DMA,
            pltpu.SemaphoreType.DMA,
            pltpu.SemaphoreType.DMA,
            pltpu.SemaphoreType.DMA,
            pltpu.SemaphoreType.DMA,
        ],
    )
    def emb_kernel(cidx_hbm, tidx_hbm, cat_tab, patch_tab, row_tab,
                   col_tab, out_hbm, cativ, tokiv, row_v, col_v, pos_v, gbuf,
                   g0, g1, g2, g3, g4, w0, w1, w2, w3, w4):
        gsem = (g0, g1, g2, g3, g4)
        wsem = (w0, w1, w2, w3, w4)
        wid = lax.axis_index("s") * NC + lax.axis_index("c")
        base = wid * BW

        pltpu.sync_copy(cidx_hbm.at[wid], cativ)
        pltpu.sync_copy(tidx_hbm.at[wid], tokiv)
        pltpu.sync_copy(row_tab, row_v)
        pltpu.sync_copy(col_tab, col_v)

        def posbody(t, carry):
            r = t // 8
            c = t % 8
            for kk in range(KD):
                s = pl.ds(kk * _LANES, _LANES)
                pos_v[t, s] = row_v[r, s] + col_v[c, s]
            return carry

        lax.fori_loop(0, T, posbody, 0)

        def issue_gather(j, p):
            @pl.when(j <= NCHK - 1)
            def _cat():
                pltpu.async_copy(
                    cat_tab.at[cativ.at[j]], gbuf.at[p], gsem[p])

            @pl.when(j >= NCHK)
            def _patch():
                pltpu.async_copy(
                    patch_tab.at[tokiv.at[j - NCHK]], gbuf.at[p], gsem[p])

        def wait_gather(p):
            pltpu.make_async_copy(
                patch_tab.at[tokiv.at[0]], gbuf.at[p], gsem[p]).wait()

        def wait_write(p):
            pltpu.make_async_copy(
                gbuf.at[p], out_hbm.at[pl.ds(0, RPS)], wsem[p]).wait()

        issue_gather(jnp.int32(0), 0)
        issue_gather(jnp.int32(1), 1)
        issue_gather(jnp.int32(2), 2)

        def iterbody(i5, carry):
            for v in range(20):
                bu = v % 5
                u = v % 4
                pv = v // 4
                i = 5 * i5 + pv
                j = NCHK * i + u
                wait_gather(bu)

                b3 = (bu + 3) % 5

                @pl.when(j >= 2)
                def _drain():
                    wait_write(b3)

                @pl.when(j <= NSTEP - 4)
                def _prefetch():
                    issue_gather(j + 3, b3)

                @pl.when(i >= 1)
                def _add():
                    posk = tuple(pos_v[i - 1, pl.ds(kk * _LANES, _LANES)]
                                 for kk in range(KD))

                    def addbody(r, pk):
                        for kk in range(KD):
                            s = pl.ds(kk * _LANES, _LANES)
                            gbuf[bu, r, s] = gbuf[bu, r, s] + pk[kk]
                        return pk

                    lax.fori_loop(0, RPS, addbody, posk)

                pltpu.async_copy(
                    gbuf.at[bu],
                    out_hbm.at[pl.ds(i * B + base + u * RPS, RPS)],
                    wsem[bu])
            return carry

        lax.fori_loop(0, OT // 5, iterbody, 0)
        wait_write((NSTEP - 2) % 5)
        wait_write((NSTEP - 1) % 5)

    return emb_kernel


def kernel(cat_idx, tokens, category_embed, patch_embed, row_embed, col_embed):
    B, T = tokens.shape
    D = patch_embed.shape[1]
    info = plsc.get_sparse_core_info()
    NW = info.num_cores * info.num_subcores
    BW = B // NW
    NCHK = BW // 128
    catr = cat_idx.astype(jnp.int32).reshape(NW, NCHK, 128)
    tokr = (tokens.astype(jnp.int32)
            .reshape(NW, NCHK, 128, T)
            .transpose(0, 3, 1, 2)
            .reshape(NW, T * NCHK, 128))
    f = _build(B, T, D, info.num_cores, info.num_subcores)
    out2d = f(catr, tokr, category_embed, patch_embed, row_embed, col_embed)
    return out2d.reshape(T + 1, B, D).transpose(1, 0, 2)


# speedup vs baseline: 1.0086x; 1.0086x over previous
import functools

import jax
import jax.numpy as jnp
from jax import lax
from jax.experimental import pallas as pl
from jax.experimental.pallas import tpu as pltpu
from jax.experimental.pallas import tpu_sc as plsc

_LANES = 16


@functools.lru_cache(maxsize=None)
def _build(B, T, D, NC, NS):
    NW = NC * NS
    BW = B // NW
    RPS = 128
    NCHK = BW // RPS
    OT = T + 1
    NSTEP = OT * NCHK
    KD = D // _LANES
    mesh = plsc.VectorSubcoreMesh(core_axis_name="c", subcore_axis_name="s")

    @functools.partial(
        pl.kernel,
        out_type=jax.ShapeDtypeStruct((OT * B, D), jnp.float32),
        mesh=mesh,
        scratch_types=[
            pltpu.VMEM((NCHK, RPS), jnp.int32),
            pltpu.VMEM((T * NCHK, RPS), jnp.int32),
            pltpu.VMEM((8, D), jnp.float32),
            pltpu.VMEM((8, D), jnp.float32),
            pltpu.VMEM((T, D), jnp.float32),
            pltpu.VMEM((NCHK, RPS, D), jnp.float32),
            pltpu.SemaphoreType.DMA,
            pltpu.SemaphoreType.DMA,
            pltpu.SemaphoreType.DMA,
            pltpu.SemaphoreType.DMA,
            pltpu.SemaphoreType.DMA,
            pltpu.SemaphoreType.DMA,
            pltpu.SemaphoreType.DMA,
            pltpu.SemaphoreType.DMA,
        ],
    )
    def emb_kernel(cidx_hbm, tidx_hbm, cat_tab, patch_tab, row_tab,
                   col_tab, out_hbm, cativ, tokiv, row_v, col_v, pos_v, gbuf,
                   g0, g1, g2, g3, w0, w1, w2, w3):
        gsem = (g0, g1, g2, g3)
        wsem = (w0, w1, w2, w3)
        wid = lax.axis_index("s") * NC + lax.axis_index("c")
        base = wid * BW

        pltpu.sync_copy(cidx_hbm.at[wid], cativ)
        pltpu.sync_copy(tidx_hbm.at[wid], tokiv)
        pltpu.sync_copy(row_tab, row_v)
        pltpu.sync_copy(col_tab, col_v)

        def posbody(t, carry):
            r = t // 8
            c = t % 8
            for kk in range(KD):
                s = pl.ds(kk * _LANES, _LANES)
                pos_v[t, s] = row_v[r, s] + col_v[c, s]
            return carry

        lax.fori_loop(0, T, posbody, 0)

        def issue_gather(j, p):
            @pl.when(j <= NCHK - 1)
            def _cat():
                pltpu.async_copy(
                    cat_tab.at[cativ.at[j]], gbuf.at[p], gsem[p])

            @pl.when(j >= NCHK)
            def _patch():
                pltpu.async_copy(
                    patch_tab.at[tokiv.at[j - NCHK]], gbuf.at[p], gsem[p])

        def wait_gather(p):
            pltpu.make_async_copy(
                patch_tab.at[tokiv.at[0]], gbuf.at[p], gsem[p]).wait()

        def wait_write(p):
            pltpu.make_async_copy(
                gbuf.at[p], out_hbm.at[pl.ds(0, RPS)], wsem[p]).wait()

        issue_gather(jnp.int32(0), 0)
        issue_gather(jnp.int32(1), 1)

        def iterbody(i, carry):
            for u in range(NCHK):
                j = NCHK * i + u
                wait_gather(u)

                u2 = (u + 2) % NCHK

                @pl.when(j >= 2)
                def _drain():
                    wait_write(u2)

                @pl.when(j <= NSTEP - 3)
                def _prefetch():
                    issue_gather(j + 2, u2)

                @pl.when(i >= 1)
                def _add():
                    posk = tuple(pos_v[i - 1, pl.ds(kk * _LANES, _LANES)]
                                 for kk in range(KD))

                    def addbody(r, pk):
                        for kk in range(KD):
                            s = pl.ds(kk * _LANES, _LANES)
                            gbuf[u, r, s] = gbuf[u, r, s] + pk[kk]
                        return pk

                    lax.fori_loop(0, RPS, addbody, posk)

                pltpu.async_copy(
                    gbuf.at[u],
                    out_hbm.at[pl.ds(i * B + base + u * RPS, RPS)],
                    wsem[u])
            return carry

        lax.fori_loop(0, OT, iterbody, 0)
        wait_write(2)
        wait_write(3)

    return emb_kernel


def kernel(cat_idx, tokens, category_embed, patch_embed, row_embed, col_embed):
    B, T = tokens.shape
    D = patch_embed.shape[1]
    info = plsc.get_sparse_core_info()
    NW = info.num_cores * info.num_subcores
    BW = B // NW
    NCHK = BW // 128
    catr = cat_idx.astype(jnp.int32).reshape(NW, NCHK, 128)
    tokr = (tokens.astype(jnp.int32)
            .reshape(NW, NCHK, 128, T)
            .transpose(0, 3, 1, 2)
            .reshape(NW, T * NCHK, 128))
    f = _build(B, T, D, info.num_cores, info.num_subcores)
    out2d = f(catr, tokr, category_embed, patch_embed, row_embed, col_embed)
    return out2d.reshape(T + 1, B, D).transpose(1, 0, 2)
